# baseline (speedup 1.0000x reference)
import functools

import jax
import jax.numpy as jnp
from jax import lax
from jax.experimental import pallas as pl
from jax.experimental.pallas import tpu as pltpu
from jax.experimental.pallas import tpu_sc as plsc

DIM = 64
SEQ = 50
NC = 2
NS = 16
NW = NC * NS
NB = 8
NIB = 4
U = 4


@functools.partial(jax.jit, static_argnames=("batches",))
def _sc_gather(table, idx, batches):
    b_per_w = batches // NW
    n_chunks = b_per_w // NB
    mesh = plsc.VectorSubcoreMesh(core_axis_name="c", subcore_axis_name="s")

    @functools.partial(
        pl.kernel,
        mesh=mesh,
        compiler_params=pltpu.CompilerParams(use_tc_tiling_on_sc=False),
        out_type=jax.ShapeDtypeStruct((batches, SEQ, DIM), jnp.float32),
        scratch_types=(
            [pltpu.VMEM((NB, SEQ), jnp.int32) for _ in range(NIB)]
            + [pltpu.VMEM((NB, SEQ, DIM), jnp.float32) for _ in range(2)]
            + [pltpu.SemaphoreType.DMA for _ in range(NIB + 4)]
        ),
    )
    def k(table_hbm, idx_hbm, out_hbm,
          iv0, iv1, iv2, iv3, rows_v0, rows_v1,
          si0, si1, si2, si3, sg0, sg1, so0, so1):
        idx_v = (iv0, iv1, iv2, iv3)
        rows_v = (rows_v0, rows_v1)
        sem_i = (si0, si1, si2, si3)
        sem_g = (sg0, sg1)
        sem_o = (so0, so1)

        wid = lax.axis_index("s") * NC + lax.axis_index("c")
        base_b = wid * b_per_w

        def idx_load(c, ib):
            pltpu.async_copy(
                idx_hbm.at[pl.ds(base_b + c * NB, NB)], idx_v[ib], sem_i[ib])

        def idx_wait(ib):
            pltpu.make_async_copy(
                idx_hbm.at[pl.ds(0, NB)], idx_v[ib], sem_i[ib]).wait()

        def fire_gathers(rb, ib):
            for r in range(NB):
                pltpu.async_copy(
                    table_hbm.at[idx_v[ib].at[r]],
                    rows_v[rb].at[r], sem_g[rb])

        def wait_gathers(rb, ib):
            for r in range(NB):
                pltpu.make_async_copy(
                    table_hbm.at[idx_v[ib].at[r]],
                    rows_v[rb].at[r], sem_g[rb]).wait()

        def out_write(c, rb):
            pltpu.async_copy(
                rows_v[rb], out_hbm.at[pl.ds(base_b + c * NB, NB)], sem_o[rb])

        def out_wait(rb):
            pltpu.make_async_copy(
                rows_v[rb], out_hbm.at[pl.ds(0, NB)], sem_o[rb]).wait()

        for c in range(3):
            idx_load(c, c)
        idx_wait(0)
        fire_gathers(0, 0)
        idx_load(3, 3)
        for c in range(1, U):
            idx_wait(c % NIB)
            if c >= 2:
                out_wait(c % 2)
            fire_gathers(c % 2, c % NIB)
            wait_gathers((c - 1) % 2, (c - 1) % NIB)
            idx_load(c + 3, (c + 3) % NIB)
            out_write(c - 1, (c - 1) % 2)

        def body(g, carry):
            c0 = U * g
            for u in range(U):
                c = c0 + u
                idx_wait(u)
                out_wait(u % 2)
                fire_gathers(u % 2, u)
                wait_gathers((u + 3) % 2, (u + 3) % NIB)
                nxt = jnp.minimum(c + 3, n_chunks - 1)
                idx_load(nxt, (u + 3) % NIB)
                out_write(c - 1, (u + 3) % 2)
            return carry

        lax.fori_loop(1, n_chunks // U, body, 0)

        last = n_chunks - 1
        wait_gathers(last % 2, last % NIB)
        out_write(last, last % 2)
        out_wait((last - 1) % 2)
        out_wait(last % 2)
        for ib in range(3):
            idx_wait(ib)

    return k(table, idx)


def kernel(bos_tensor, table):
    batches = bos_tensor.shape[0]
    return _sc_gather(table, bos_tensor.astype(jnp.int32), batches)


# speedup vs baseline: 1.0117x; 1.0117x over previous
import functools

import jax
import jax.numpy as jnp
from jax import lax
from jax.experimental import pallas as pl
from jax.experimental.pallas import tpu as pltpu
from jax.experimental.pallas import tpu_sc as plsc

DIM = 64
NC = 2
NS = 16
NW = NC * NS
IDXW = 128
KPC = 5
CHUNK = KPC * IDXW
NIB = 4
U = 4


@functools.partial(jax.jit, static_argnames=("b_total",))
def _sc_gather(table, idx1d, b_total):
    b_per_w = b_total // NW
    n_chunks = b_per_w // CHUNK
    mesh = plsc.VectorSubcoreMesh(core_axis_name="c", subcore_axis_name="s")

    @functools.partial(
        pl.kernel,
        mesh=mesh,
        compiler_params=pltpu.CompilerParams(use_tc_tiling_on_sc=False),
        out_type=jax.ShapeDtypeStruct((b_total, DIM), jnp.float32),
        scratch_types=(
            [pltpu.VMEM((CHUNK,), jnp.int32) for _ in range(NIB)]
            + [pltpu.VMEM((CHUNK, DIM), jnp.float32) for _ in range(2)]
            + [pltpu.SemaphoreType.DMA for _ in range(NIB + 4)]
        ),
    )
    def k(table_hbm, idx_hbm, out_hbm,
          iv0, iv1, iv2, iv3, rows_v0, rows_v1,
          si0, si1, si2, si3, sg0, sg1, so0, so1):
        idx_v = (iv0, iv1, iv2, iv3)
        rows_v = (rows_v0, rows_v1)
        sem_i = (si0, si1, si2, si3)
        sem_g = (sg0, sg1)
        sem_o = (so0, so1)

        wid = lax.axis_index("s") * NC + lax.axis_index("c")
        base_row = wid * b_per_w

        def idx_load(c, ib):
            pltpu.async_copy(
                idx_hbm.at[pl.ds(base_row + c * CHUNK, CHUNK)],
                idx_v[ib], sem_i[ib])

        def idx_wait(ib):
            pltpu.make_async_copy(
                idx_hbm.at[pl.ds(0, CHUNK)], idx_v[ib], sem_i[ib]).wait()

        def fire_gathers(rb, ib):
            for j in range(KPC):
                pltpu.async_copy(
                    table_hbm.at[idx_v[ib].at[pl.ds(j * IDXW, IDXW)]],
                    rows_v[rb].at[pl.ds(j * IDXW, IDXW)],
                    sem_g[rb])

        def wait_gathers(rb, ib):
            for j in range(KPC):
                pltpu.make_async_copy(
                    table_hbm.at[idx_v[ib].at[pl.ds(j * IDXW, IDXW)]],
                    rows_v[rb].at[pl.ds(j * IDXW, IDXW)],
                    sem_g[rb]).wait()

        def out_write(c, rb):
            pltpu.async_copy(
                rows_v[rb], out_hbm.at[pl.ds(base_row + c * CHUNK, CHUNK)],
                sem_o[rb])

        def out_wait(rb):
            pltpu.make_async_copy(
                rows_v[rb], out_hbm.at[pl.ds(0, CHUNK)], sem_o[rb]).wait()

        for c in range(3):
            idx_load(c, c)
        idx_wait(0)
        fire_gathers(0, 0)
        idx_load(3, 3)
        for c in range(1, U):
            idx_wait(c % NIB)
            if c >= 2:
                out_wait(c % 2)
            fire_gathers(c % 2, c % NIB)
            wait_gathers((c - 1) % 2, (c - 1) % NIB)
            idx_load(c + 3, (c + 3) % NIB)
            out_write(c - 1, (c - 1) % 2)

        def body(g, carry):
            c0 = U * g
            for u in range(U):
                c = c0 + u
                idx_wait(u)
                out_wait(u % 2)
                fire_gathers(u % 2, u)
                wait_gathers((u + 3) % 2, (u + 3) % NIB)
                nxt = jnp.minimum(c + 3, n_chunks - 1)
                idx_load(nxt, (u + 3) % NIB)
                out_write(c - 1, (u + 3) % 2)
            return carry

        lax.fori_loop(1, n_chunks // U, body, 0)

        last = n_chunks - 1
        wait_gathers(last % 2, last % NIB)
        out_write(last, last % 2)
        out_wait((last - 1) % 2)
        out_wait(last % 2)
        for ib in range(3):
            idx_wait(ib)

    return k(table, idx1d)


def kernel(bos_tensor, table):
    b, l = bos_tensor.shape
    b_total = b * l
    idx1d = bos_tensor.astype(jnp.int32).reshape(b_total)
    out = _sc_gather(table, idx1d, b_total)
    return out.reshape(b, l, DIM)
